# baseline (speedup 1.0000x reference)
import functools

import jax
import jax.numpy as jnp
from jax import lax
from jax.experimental import pallas as pl
from jax.experimental.pallas import tpu as pltpu
from jax.experimental.pallas import tpu_sc as plsc

B, H, W, D = 4, 32, 32, 256
DEPTH = 4
K = 8192
M = B * H * W
TM = 2048
G = M // TM
CK = 1024
NSLC = CK // 128

NC, NS = 2, 16
NWORK = NC * NS
TPW = M // NWORK


def _argmin_body(r_ref, cb_ref, idx_ref, ss_ref, cbsq_ref):
    r = r_ref[...]
    r2 = r + r
    in_sq = jnp.sum(r * r, axis=1, keepdims=True)
    ss_ref[...] = jnp.zeros((1, 1, 128), jnp.float32) + jnp.sum(in_sq)
    lane = lax.broadcasted_iota(jnp.int32, (TM, 128), 1)

    @pl.when(pl.program_id(0) == 0)
    def _():
        for c in range(K // CK):
            cbc = cb_ref[pl.ds(c * CK, CK), :]
            cbsq_ref[:, pl.ds(c * CK, CK)] = jnp.sum(cbc * cbc, axis=1)[None, :]

    def mm_chunk(c):
        cbc = cb_ref[pl.ds(c * CK, CK), :]
        return lax.dot_general(r2, cbc, (((1,), (1,)), ((), ())),
                               preferred_element_type=jnp.float32)

    rm = jnp.full((TM, 128), jnp.inf, jnp.float32)
    rmi = jnp.zeros((TM, 128), jnp.int32)
    for c in range(K // CK):
        mm2 = mm_chunk(c)
        cb_sq = cbsq_ref[:, c * CK:(c + 1) * CK]
        for j in range(NSLC):
            sl = slice(j * 128, (j + 1) * 128)
            dj = (in_sq + cb_sq[:, sl]) - mm2[:, sl]
            slot = c * NSLC + j
            better = dj < rm
            rm = jnp.minimum(dj, rm)
            rmi = jnp.where(better, slot, rmi)
    best = jnp.min(rm, axis=1, keepdims=True)
    idx_ref[...] = jnp.min(jnp.where(rm == best, rmi * 128 + lane, K),
                           axis=1, keepdims=True)


def _tc_argmin(r_flat, ncb):
    rows = r_flat.shape[0]
    grid = rows // TM
    return pl.pallas_call(
        _argmin_body,
        grid=(grid,),
        in_specs=[
            pl.BlockSpec((TM, D), lambda m: (m, 0)),
            pl.BlockSpec((K, D), lambda m: (0, 0)),
        ],
        out_specs=[
            pl.BlockSpec((TM, 1), lambda m: (m, 0)),
            pl.BlockSpec((1, 1, 128), lambda m: (m, 0, 0)),
        ],
        out_shape=[
            jax.ShapeDtypeStruct((rows, 1), jnp.int32),
            jax.ShapeDtypeStruct((grid, 1, 128), jnp.float32),
        ],
        scratch_shapes=[pltpu.VMEM((1, K), jnp.float32)],
    )(r_flat, ncb)


@functools.cache
def _make_sc_update(rows, last):
    TPW = rows // NWORK
    mesh = plsc.VectorSubcoreMesh(core_axis_name="c", subcore_axis_name="s")
    scratch = [
        pltpu.VMEM((TPW,), jnp.int32),
        pltpu.VMEM((TPW, D), jnp.float32),
        pltpu.VMEM((TPW, D), jnp.float32),
        pltpu.SemaphoreType.DMA,
    ]
    if last:
        out_type = [jax.ShapeDtypeStruct((rows, D), jnp.float32),
                    jax.ShapeDtypeStruct((NWORK, 16), jnp.float32)]
        scratch = [pltpu.VMEM((TPW, D), jnp.float32),
                   pltpu.VMEM((16,), jnp.float32)] + scratch
    else:
        out_type = jax.ShapeDtypeStruct((rows, D), jnp.float32)

    if not last:
        @functools.partial(pl.kernel, mesh=mesh, out_type=out_type,
                           scratch_types=scratch)
        def sc_update(cb_hbm, idx_hbm, r_hbm, rout_hbm,
                      idx_v, q_v, r_v, sem):
            wid = lax.axis_index("s") * NC + lax.axis_index("c")
            base = wid * TPW
            pltpu.sync_copy(idx_hbm.at[pl.ds(base, TPW)], idx_v)
            gather = pltpu.async_copy(cb_hbm.at[idx_v], q_v, sem)
            pltpu.sync_copy(r_hbm.at[pl.ds(base, TPW), :], r_v)
            gather.wait()

            def token_body(t, carry):
                for c in range(D // 16):
                    sl = pl.ds(c * 16, 16)
                    r_v[t, sl] = r_v[t, sl] - q_v[t, sl]
                return carry

            lax.fori_loop(0, TPW, token_body, 0)
            pltpu.sync_copy(r_v, rout_hbm.at[pl.ds(base, TPW), :])

        return sc_update

    @functools.partial(pl.kernel, mesh=mesh, out_type=out_type,
                       scratch_types=scratch)
    def sc_update_last(cb_hbm, idx_hbm, r_hbm, x_hbm, qout_hbm, ss_hbm,
                       x_v, acc_v, idx_v, q_v, r_v, sem):
        wid = lax.axis_index("s") * NC + lax.axis_index("c")
        base = wid * TPW
        pltpu.sync_copy(idx_hbm.at[pl.ds(base, TPW)], idx_v)
        gather = pltpu.async_copy(cb_hbm.at[idx_v], q_v, sem)
        pltpu.sync_copy(r_hbm.at[pl.ds(base, TPW), :], r_v)
        pltpu.sync_copy(x_hbm.at[pl.ds(base, TPW), :], x_v)
        gather.wait()

        def token_body(t, acc):
            for c in range(D // 16):
                sl = pl.ds(c * 16, 16)
                v = r_v[t, sl] - q_v[t, sl]
                r_v[t, sl] = x_v[t, sl] - v
                acc = acc + v * v
            return acc

        acc = lax.fori_loop(0, TPW, token_body, jnp.zeros((16,), jnp.float32))
        acc_v[...] = acc
        pltpu.sync_copy(r_v, qout_hbm.at[pl.ds(base, TPW), :])
        pltpu.sync_copy(acc_v, ss_hbm.at[wid])

    return sc_update_last


def kernel(x, codebooks):
    x_flat = x.reshape(M, D)
    r = x_flat
    codes = []
    loss_parts = []
    for i in range(DEPTH):
        idx_col, ss_tc = _tc_argmin(r, codebooks[i])
        idx = idx_col.reshape(M)
        if i > 0:
            loss_parts.append(jnp.sum(ss_tc[:, 0, 0]))
        if i < DEPTH - 1:
            r = _make_sc_update(M, False)(codebooks[i], idx, r)
        else:
            quants_flat, ss_sc = _make_sc_update(M, True)(
                codebooks[i], idx, r, x_flat)
            loss_parts.append(jnp.sum(ss_sc))
        codes.append(idx)
    losses = jnp.stack(loss_parts) / (M * D)
    commitment_loss = jnp.mean(losses)
    quants_trunc = quants_flat.reshape(x.shape)
    codes_arr = jnp.stack(codes, axis=-1).reshape(B, H, W, DEPTH)
    return quants_trunc, commitment_loss, codes_arr


# speedup vs baseline: 1.0491x; 1.0491x over previous
import functools

import jax
import jax.numpy as jnp
from jax import lax
from jax.experimental import pallas as pl
from jax.experimental.pallas import tpu as pltpu
from jax.experimental.pallas import tpu_sc as plsc

B, H, W, D = 4, 32, 32, 256
DEPTH = 4
K = 8192
M = B * H * W
TM = 1024
G = M // TM
CK = 1024
NSLC = CK // 128

NC, NS = 2, 16
NWORK = NC * NS
TPW = M // NWORK


def _make_argmin_body(nq, emit_r):
    def body(*refs):
        x_ref = refs[0]
        q_refs = refs[1:1 + nq]
        cb_ref = refs[1 + nq]
        idx_ref = refs[2 + nq]
        ss_ref = refs[3 + nq]
        rout_ref = refs[4 + nq] if emit_r else None
        cbsq_ref = refs[-1]

        r = x_ref[...]
        for q_ref in q_refs:
            r = r - q_ref[...]
        if emit_r:
            rout_ref[...] = r
        r2 = r + r
        in_sq = jnp.sum(r * r, axis=1, keepdims=True)
        ss_ref[...] = jnp.zeros((1, 1, 128), jnp.float32) + jnp.sum(in_sq)
        lane = lax.broadcasted_iota(jnp.int32, (TM, 128), 1)

        @pl.when(pl.program_id(0) == 0)
        def _():
            for c in range(K // CK):
                cbc = cb_ref[pl.ds(c * CK, CK), :]
                cbsq_ref[:, pl.ds(c * CK, CK)] = (
                    jnp.sum(cbc * cbc, axis=1)[None, :])

        def mm_chunk(c):
            cbc = cb_ref[pl.ds(c * CK, CK), :]
            return lax.dot_general(r2, cbc, (((1,), (1,)), ((), ())),
                                   preferred_element_type=jnp.float32)

        rm = jnp.full((TM, 128), jnp.inf, jnp.float32)
        rmi = jnp.zeros((TM, 128), jnp.int32)
        for c in range(K // CK):
            mm2 = mm_chunk(c)
            cb_sq = cbsq_ref[:, c * CK:(c + 1) * CK]
            for j in range(NSLC):
                sl = slice(j * 128, (j + 1) * 128)
                dj = (in_sq + cb_sq[:, sl]) - mm2[:, sl]
                slot = c * NSLC + j
                better = dj < rm
                rm = jnp.minimum(dj, rm)
                rmi = jnp.where(better, slot, rmi)
        best = jnp.min(rm, axis=1, keepdims=True)
        idx_ref[...] = jnp.min(jnp.where(rm == best, rmi * 128 + lane, K),
                               axis=1, keepdims=True)

    return body


@functools.cache
def _tc_argmin(nq, emit_r):
    mat_spec = pl.BlockSpec((TM, D), lambda m: (m, 0))
    out_specs = [
        pl.BlockSpec((TM, 1), lambda m: (m, 0)),
        pl.BlockSpec((1, 1, 128), lambda m: (m, 0, 0)),
    ]
    out_shape = [
        jax.ShapeDtypeStruct((M, 1), jnp.int32),
        jax.ShapeDtypeStruct((G, 1, 128), jnp.float32),
    ]
    if emit_r:
        out_specs.append(mat_spec)
        out_shape.append(jax.ShapeDtypeStruct((M, D), jnp.float32))
    return pl.pallas_call(
        _make_argmin_body(nq, emit_r),
        grid=(G,),
        in_specs=[mat_spec] * (1 + nq) + [pl.BlockSpec((K, D), lambda m: (0, 0))],
        out_specs=out_specs,
        out_shape=out_shape,
        scratch_shapes=[pltpu.VMEM((1, K), jnp.float32)],
    )


@functools.cache
def _make_sc_update(rows, last):
    TPW = rows // NWORK
    mesh = plsc.VectorSubcoreMesh(core_axis_name="c", subcore_axis_name="s")
    scratch = [
        pltpu.VMEM((TPW,), jnp.int32),
        pltpu.VMEM((TPW, D), jnp.float32),
        pltpu.VMEM((TPW, D), jnp.float32),
        pltpu.SemaphoreType.DMA,
    ]
    if last:
        out_type = [jax.ShapeDtypeStruct((rows, D), jnp.float32),
                    jax.ShapeDtypeStruct((NWORK, 16), jnp.float32)]
        scratch = [pltpu.VMEM((TPW, D), jnp.float32),
                   pltpu.VMEM((16,), jnp.float32)] + scratch
    else:
        out_type = jax.ShapeDtypeStruct((rows, D), jnp.float32)

    if not last:
        @functools.partial(pl.kernel, mesh=mesh, out_type=out_type,
                           scratch_types=scratch)
        def sc_gather(cb_hbm, idx_hbm, qout_hbm, idx_v, q_v, r_v, sem):
            del r_v
            wid = lax.axis_index("s") * NC + lax.axis_index("c")
            base = wid * TPW
            pltpu.sync_copy(idx_hbm.at[pl.ds(base, TPW)], idx_v)
            pltpu.async_copy(cb_hbm.at[idx_v], q_v, sem).wait()
            pltpu.sync_copy(q_v, qout_hbm.at[pl.ds(base, TPW), :])

        return sc_gather

    @functools.partial(pl.kernel, mesh=mesh, out_type=out_type,
                       scratch_types=scratch)
    def sc_update_last(cb_hbm, idx_hbm, r_hbm, x_hbm, qout_hbm, ss_hbm,
                       x_v, acc_v, idx_v, q_v, r_v, sem):
        wid = lax.axis_index("s") * NC + lax.axis_index("c")
        base = wid * TPW
        pltpu.sync_copy(idx_hbm.at[pl.ds(base, TPW)], idx_v)
        gather = pltpu.async_copy(cb_hbm.at[idx_v], q_v, sem)
        pltpu.sync_copy(r_hbm.at[pl.ds(base, TPW), :], r_v)
        pltpu.sync_copy(x_hbm.at[pl.ds(base, TPW), :], x_v)
        gather.wait()

        def token_body(t, acc):
            for c in range(D // 16):
                sl = pl.ds(c * 16, 16)
                v = r_v[t, sl] - q_v[t, sl]
                r_v[t, sl] = x_v[t, sl] - v
                acc = acc + v * v
            return acc

        acc = lax.fori_loop(0, TPW, token_body, jnp.zeros((16,), jnp.float32))
        acc_v[...] = acc
        pltpu.sync_copy(r_v, qout_hbm.at[pl.ds(base, TPW), :])
        pltpu.sync_copy(acc_v, ss_hbm.at[wid])

    return sc_update_last


def kernel(x, codebooks):
    x_flat = x.reshape(M, D)
    codes = []
    loss_parts = []
    qs = []
    for i in range(DEPTH):
        last = i == DEPTH - 1
        outs = _tc_argmin(i, last)(x_flat, *qs, codebooks[i])
        idx = outs[0].reshape(M)
        if i > 0:
            loss_parts.append(jnp.sum(outs[1][:, 0, 0]))
        if not last:
            qs.append(_make_sc_update(M, False)(codebooks[i], idx))
        else:
            quants_flat, ss_sc = _make_sc_update(M, True)(
                codebooks[i], idx, outs[2], x_flat)
            loss_parts.append(jnp.sum(ss_sc))
        codes.append(idx)
    losses = jnp.stack(loss_parts) / (M * D)
    commitment_loss = jnp.mean(losses)
    quants_trunc = quants_flat.reshape(x.shape)
    codes_arr = jnp.stack(codes, axis=-1).reshape(B, H, W, DEPTH)
    return quants_trunc, commitment_loss, codes_arr
